# baseline (speedup 1.0000x reference)
import functools

import jax
import jax.numpy as jnp
from jax import lax
from jax.experimental import pallas as pl
from jax.experimental.pallas import tpu as pltpu
from jax.experimental.pallas import tpu_sc as plsc

_NIN = 1024
_NOUT = 1024
_DEPTH = 10
_NNODES = 1023
_B = 8192
_HALF = _NOUT // 2

_NC = 2
_NS = 16
_NW = _NC * _NS
_BPW = _B // _NW
_CHUNK = 16
_NCHUNKS = _BPW // _CHUNK
_LANES = 16
_DSTEPS = _HALF // _LANES

_NDENSE = 8
_NDNODE = 256
_NDEEP = _DEPTH - _NDENSE
_CGD = _CHUNK * _NDEEP


_TBLK = 1024
_YBLK = 128


def _pack_halves(m):
    a = lax.bitcast_convert_type(m[:, :_HALF].astype(jnp.bfloat16),
                                 jnp.uint16).astype(jnp.int32)
    b = lax.bitcast_convert_type(m[:, _HALF:].astype(jnp.bfloat16),
                                 jnp.uint16).astype(jnp.int32)
    return a | (b << 16)


def _router_body(x_ref, w_ref, ysh_ref, ypk_ref,
                 par_ref, lam_ref, ytab_ref):
    x = x_ref[...]
    w = w_ref[...]
    lam = lax.dot_general(x, w, (((1,), (1,)), ((), ())),
                          preferred_element_type=jnp.float32)
    branch = (lam > 0).astype(jnp.float32)
    jj = lax.broadcasted_iota(jnp.int32, (_DEPTH, _NDNODE), 0)
    nrow = lax.broadcasted_iota(jnp.int32, (1, _NDNODE), 1)
    lev = jnp.zeros((1, _NDNODE), jnp.int32)
    for i in range(1, _NDENSE + 1):
        lev = lev + (nrow + 1 >= (1 << i)).astype(jnp.int32)
    p = nrow + 1 - (1 << lev)
    pc = jnp.zeros((1, _NDNODE), jnp.int32)
    for k in range(_NDENSE):
        pc = pc + ((p >> k) & 1)
    cvec = (lev - pc).astype(jnp.float32)
    levf = lev.astype(jnp.float32)
    sh = jnp.maximum(lev - 1 - jj, 0)
    r = (jnp.broadcast_to(p, (_DEPTH, _NDNODE)) >> sh) & 1
    q = jnp.where(jj < lev, (2 * r - 1).astype(jnp.float32), 0.0)
    rr = jnp.where(jj == lev, 1.0, 0.0)
    score = lax.dot_general(branch, q, (((1,), (0,)), ((), ())),
                            preferred_element_type=jnp.float32) + cvec
    lamb = lax.dot_general(lam, rr, (((1,), (0,)), ((), ())),
                           preferred_element_type=jnp.float32)
    s = jnp.where(score == levf, lamb, 0.0)
    rows = lax.broadcasted_iota(jnp.int32, (_NDNODE, 1), 0)
    ysh = jnp.where(rows < _NDNODE - 1, ysh_ref[...], 0.0)
    partial = lax.dot_general(s.astype(jnp.bfloat16),
                              ysh.astype(jnp.bfloat16),
                              (((1,), (0,)), ((), ())),
                              preferred_element_type=jnp.float32)
    par_ref[...] = _pack_halves(partial)
    lam_ref[...] = lam.T
    ytab_ref[...] = _pack_halves(ypk_ref[...])


def _router(x2, w_sel, y):
    grid = (_B // _TBLK,)
    return pl.pallas_call(
        _router_body,
        grid=grid,
        in_specs=[
            pl.BlockSpec((_TBLK, _NIN), lambda i: (i, 0)),
            pl.BlockSpec((_DEPTH, _NIN), lambda i: (0, 0)),
            pl.BlockSpec((_NDNODE, _NOUT), lambda i: (0, 0)),
            pl.BlockSpec((_YBLK, _NOUT), lambda i: (i, 0)),
        ],
        out_specs=[
            pl.BlockSpec((_TBLK, _HALF), lambda i: (i, 0)),
            pl.BlockSpec((_DEPTH, _TBLK), lambda i: (0, i)),
            pl.BlockSpec((_YBLK, _HALF), lambda i: (i, 0)),
        ],
        out_shape=[
            jax.ShapeDtypeStruct((_B, _HALF), jnp.int32),
            jax.ShapeDtypeStruct((_DEPTH, _B), jnp.float32),
            jax.ShapeDtypeStruct((_NNODES, _HALF), jnp.int32),
        ],
    )(x2, w_sel, y, y)


def _issue_gather(c, ytab_hbm, dp_all, rows_v, sem):
    pltpu.async_copy(ytab_hbm.at[dp_all.at[pl.ds(c * _CGD, _CGD)]], rows_v,
                     sem)


def _wait_gather(c, ytab_hbm, dp_all, rows_v, sem):
    pltpu.make_async_copy(
        ytab_hbm.at[dp_all.at[pl.ds(c * _CGD, _CGD)]], rows_v, sem).wait()


def _compute_chunk(c, lam_all, rows_v, par_v, out_v):
    lamvecs = [lam_all[_NDENSE + i, pl.ds(c * _CHUNK, _LANES)]
               for i in range(_NDEEP)]
    neg16 = jnp.int32(-65536)
    for t in range(_CHUNK):
        w = [jnp.full((_LANES,), lamvecs[i][t], jnp.float32)
             for i in range(_NDEEP)]

        @plsc.parallel_loop(0, _DSTEPS, unroll=8)
        def _d_body(d):
            sl = pl.ds(d * _LANES, _LANES)
            pv = par_v[t, sl]
            acc_lo = lax.bitcast_convert_type(pv << 16, jnp.float32)
            acc_hi = lax.bitcast_convert_type(pv & neg16, jnp.float32)
            for i in range(_NDEEP):
                v = rows_v[i * _CHUNK + t, sl]
                lo = lax.bitcast_convert_type(v << 16, jnp.float32)
                hi = lax.bitcast_convert_type(v, jnp.float32)
                acc_lo = acc_lo + w[i] * lo
                acc_hi = acc_hi + w[i] * hi
            out_v[t, sl] = acc_lo
            out_v[t, pl.ds(_HALF + d * _LANES, _LANES)] = acc_hi


def _combine_body(lamt_hbm, ytab_hbm, par_hbm, out_hbm,
                  lam_all, dp_all, rows_v0, rows_v1, par_v0, par_v1,
                  out_v0, out_v1, semg0, semg1, semp0, semp1, semo0, semo1):
    wid = lax.axis_index("s") * _NC + lax.axis_index("c")
    base = wid * _BPW
    for l in range(_DEPTH):
        pltpu.async_copy(lamt_hbm.at[l, pl.ds(base, _BPW)], lam_all.at[l],
                         semg0)
    for l in range(_DEPTH):
        pltpu.make_async_copy(lamt_hbm.at[l, pl.ds(base, _BPW)],
                              lam_all.at[l], semg0).wait()
    for g in range(_BPW // _LANES):
        sl = pl.ds(g * _LANES, _LANES)
        one = jnp.int32(1)
        zero = jnp.int32(0)
        p8 = jnp.zeros((_LANES,), jnp.int32)
        for l in range(_NDENSE):
            b = jnp.where(lam_all[l, sl] > 0.0, one, zero)
            p8 = p8 + (b << (_NDENSE - 1 - l))
        b8 = jnp.where(lam_all[_NDENSE, sl] > 0.0, one, zero)
        idx8 = p8 + jnp.int32(_NDNODE - 1)
        idx9 = idx8 * 2 + 1 + b8
        dp_all[pl.ds(g * _LANES * _NDEEP, _LANES)] = idx8
        dp_all[pl.ds(g * _LANES * _NDEEP + _LANES, _LANES)] = idx9

    def par_slice(c):
        return par_hbm.at[pl.ds(base + c * _CHUNK, _CHUNK)]

    def out_slice(c):
        return out_hbm.at[pl.ds(base + c * _CHUNK, _CHUNK)]

    _issue_gather(0, ytab_hbm, dp_all, rows_v0, semg0)
    pltpu.async_copy(par_slice(0), par_v0, semp0)

    def loop(c2, carry):
        c = c2 * 2
        _issue_gather(c + 1, ytab_hbm, dp_all, rows_v1, semg1)
        pltpu.async_copy(par_slice(c + 1), par_v1, semp1)
        _wait_gather(c, ytab_hbm, dp_all, rows_v0, semg0)
        pltpu.make_async_copy(par_slice(c), par_v0, semp0).wait()

        @pl.when(c2 > 0)
        def _():
            pltpu.make_async_copy(out_v0, out_slice(c - 2), semo0).wait()

        _compute_chunk(c, lam_all, rows_v0, par_v0, out_v0)
        pltpu.async_copy(out_v0, out_slice(c), semo0)

        @pl.when(c2 < _NCHUNKS // 2 - 1)
        def _():
            _issue_gather(c + 2, ytab_hbm, dp_all, rows_v0, semg0)
            pltpu.async_copy(par_slice(c + 2), par_v0, semp0)

        _wait_gather(c + 1, ytab_hbm, dp_all, rows_v1, semg1)
        pltpu.make_async_copy(par_slice(c + 1), par_v1, semp1).wait()

        @pl.when(c2 > 0)
        def _():
            pltpu.make_async_copy(out_v1, out_slice(c - 1), semo1).wait()

        _compute_chunk(c + 1, lam_all, rows_v1, par_v1, out_v1)
        pltpu.async_copy(out_v1, out_slice(c + 1), semo1)
        return carry

    lax.fori_loop(0, _NCHUNKS // 2, loop, 0)
    pltpu.make_async_copy(out_v0, out_slice(_NCHUNKS - 2), semo0).wait()
    pltpu.make_async_copy(out_v1, out_slice(_NCHUNKS - 1), semo1).wait()


def _combine(lamt, ytab_packed, partial):
    mesh = plsc.VectorSubcoreMesh(core_axis_name="c", subcore_axis_name="s",
                                  num_cores=_NC, num_subcores=_NS)
    f = pl.kernel(
        _combine_body,
        out_type=jax.ShapeDtypeStruct((_B, _NOUT), jnp.float32),
        mesh=mesh,
        scratch_types=[
            pltpu.VMEM((_DEPTH, _BPW), jnp.float32),
            pltpu.VMEM((_BPW * _NDEEP,), jnp.int32),
            pltpu.VMEM((_CGD, _HALF), jnp.int32),
            pltpu.VMEM((_CGD, _HALF), jnp.int32),
            pltpu.VMEM((_CHUNK, _HALF), jnp.int32),
            pltpu.VMEM((_CHUNK, _HALF), jnp.int32),
            pltpu.VMEM((_CHUNK, _NOUT), jnp.float32),
            pltpu.VMEM((_CHUNK, _NOUT), jnp.float32),
            pltpu.SemaphoreType.DMA,
            pltpu.SemaphoreType.DMA,
            pltpu.SemaphoreType.DMA,
            pltpu.SemaphoreType.DMA,
            pltpu.SemaphoreType.DMA,
            pltpu.SemaphoreType.DMA,
        ],
    )
    return f(lamt, ytab_packed, partial)


def kernel(x, W_sel, Y):
    orig_shape = x.shape
    x2 = x.reshape(-1, _NIN) if x.ndim == 3 else x
    partial, lamt, ytab = _router(x2, W_sel, Y)
    y = _combine(lamt, ytab, partial)
    if orig_shape[1] != _NIN:
        y = y.reshape(orig_shape[0], orig_shape[1], _NOUT)
    return y


# speedup vs baseline: 1.1387x; 1.1387x over previous
import functools

import jax
import jax.numpy as jnp
from jax import lax
from jax.experimental import pallas as pl
from jax.experimental.pallas import tpu as pltpu
from jax.experimental.pallas import tpu_sc as plsc

_NIN = 1024
_NOUT = 1024
_DEPTH = 10
_NNODES = 1023
_B = 8192
_HALF = _NOUT // 2

_NC = 2
_NS = 16
_NW = _NC * _NS
_BPW = _B // _NW
_CHUNK = 16
_NCHUNKS = _BPW // _CHUNK
_LANES = 16
_DSTEPS = _HALF // _LANES

_NDENSE = 9
_NDNODE = 512
_NDEEP = _DEPTH - _NDENSE
_CGD = _CHUNK * _NDEEP


_TBLK = 1024
_YBLK = 128


def _pack_halves(m):
    a = lax.bitcast_convert_type(m[:, :_HALF].astype(jnp.bfloat16),
                                 jnp.uint16).astype(jnp.int32)
    b = lax.bitcast_convert_type(m[:, _HALF:].astype(jnp.bfloat16),
                                 jnp.uint16).astype(jnp.int32)
    return a | (b << 16)


def _router_body(x_ref, w_ref, ysh_ref, ypk_ref,
                 par_ref, lam_ref, ytab_ref):
    x = x_ref[...]
    w = w_ref[...]
    lam = lax.dot_general(x, w, (((1,), (1,)), ((), ())),
                          preferred_element_type=jnp.float32)
    branch = (lam > 0).astype(jnp.float32)
    jj = lax.broadcasted_iota(jnp.int32, (_DEPTH, _NDNODE), 0)
    nrow = lax.broadcasted_iota(jnp.int32, (1, _NDNODE), 1)
    lev = jnp.zeros((1, _NDNODE), jnp.int32)
    for i in range(1, _NDENSE + 1):
        lev = lev + (nrow + 1 >= (1 << i)).astype(jnp.int32)
    p = nrow + 1 - (1 << lev)
    pc = jnp.zeros((1, _NDNODE), jnp.int32)
    for k in range(_NDENSE):
        pc = pc + ((p >> k) & 1)
    cvec = (lev - pc).astype(jnp.float32)
    levf = lev.astype(jnp.float32)
    sh = jnp.maximum(lev - 1 - jj, 0)
    r = (jnp.broadcast_to(p, (_DEPTH, _NDNODE)) >> sh) & 1
    q = jnp.where(jj < lev, (2 * r - 1).astype(jnp.float32), 0.0)
    rr = jnp.where(jj == lev, 1.0, 0.0)
    score = lax.dot_general(branch, q, (((1,), (0,)), ((), ())),
                            preferred_element_type=jnp.float32) + cvec
    lamb = lax.dot_general(lam, rr, (((1,), (0,)), ((), ())),
                           preferred_element_type=jnp.float32)
    s = jnp.where(score == levf, lamb, 0.0)
    rows = lax.broadcasted_iota(jnp.int32, (_NDNODE, 1), 0)
    ysh = jnp.where(rows < _NDNODE - 1, ysh_ref[...], 0.0)
    partial = lax.dot_general(s.astype(jnp.bfloat16),
                              ysh.astype(jnp.bfloat16),
                              (((1,), (0,)), ((), ())),
                              preferred_element_type=jnp.float32)
    par_ref[...] = _pack_halves(partial)
    lam_ref[...] = lam.T
    ytab_ref[...] = _pack_halves(ypk_ref[...])


def _router(x2, w_sel, y):
    grid = (_B // _TBLK,)
    return pl.pallas_call(
        _router_body,
        grid=grid,
        in_specs=[
            pl.BlockSpec((_TBLK, _NIN), lambda i: (i, 0)),
            pl.BlockSpec((_DEPTH, _NIN), lambda i: (0, 0)),
            pl.BlockSpec((_NDNODE, _NOUT), lambda i: (0, 0)),
            pl.BlockSpec((_YBLK, _NOUT), lambda i: (i, 0)),
        ],
        out_specs=[
            pl.BlockSpec((_TBLK, _HALF), lambda i: (i, 0)),
            pl.BlockSpec((_DEPTH, _TBLK), lambda i: (0, i)),
            pl.BlockSpec((_YBLK, _HALF), lambda i: (i, 0)),
        ],
        out_shape=[
            jax.ShapeDtypeStruct((_B, _HALF), jnp.int32),
            jax.ShapeDtypeStruct((_DEPTH, _B), jnp.float32),
            jax.ShapeDtypeStruct((_NNODES, _HALF), jnp.int32),
        ],
    )(x2, w_sel, y, y)


def _issue_gather(c, ytab_hbm, dp_all, rows_v, sem):
    pltpu.async_copy(ytab_hbm.at[dp_all.at[pl.ds(c * _CGD, _CGD)]], rows_v,
                     sem)


def _wait_gather(c, ytab_hbm, dp_all, rows_v, sem):
    pltpu.make_async_copy(
        ytab_hbm.at[dp_all.at[pl.ds(c * _CGD, _CGD)]], rows_v, sem).wait()


def _compute_chunk(c, lam_all, rows_v, par_v, out_v):
    lamvecs = [lam_all[_NDENSE + i, pl.ds(c * _CHUNK, _LANES)]
               for i in range(_NDEEP)]
    neg16 = jnp.int32(-65536)
    for t in range(_CHUNK):
        w = [jnp.full((_LANES,), lamvecs[i][t], jnp.float32)
             for i in range(_NDEEP)]

        @plsc.parallel_loop(0, _DSTEPS, unroll=4)
        def _d_body(d):
            sl = pl.ds(d * _LANES, _LANES)
            pv = par_v[t, sl]
            acc_lo = lax.bitcast_convert_type(pv << 16, jnp.float32)
            acc_hi = lax.bitcast_convert_type(pv & neg16, jnp.float32)
            for i in range(_NDEEP):
                v = rows_v[i * _CHUNK + t, sl]
                lo = lax.bitcast_convert_type(v << 16, jnp.float32)
                hi = lax.bitcast_convert_type(v, jnp.float32)
                acc_lo = acc_lo + w[i] * lo
                acc_hi = acc_hi + w[i] * hi
            out_v[t, sl] = acc_lo
            out_v[t, pl.ds(_HALF + d * _LANES, _LANES)] = acc_hi


def _combine_body(lamt_hbm, ytab_hbm, par_hbm, out_hbm,
                  lam_all, dp_all, rows_v0, rows_v1, par_v0, par_v1,
                  out_v0, out_v1, semg0, semg1, semp0, semp1, semo0, semo1):
    wid = lax.axis_index("s") * _NC + lax.axis_index("c")
    base = wid * _BPW
    for l in range(_DEPTH):
        pltpu.async_copy(lamt_hbm.at[l, pl.ds(base, _BPW)], lam_all.at[l],
                         semg0)
    for l in range(_DEPTH):
        pltpu.make_async_copy(lamt_hbm.at[l, pl.ds(base, _BPW)],
                              lam_all.at[l], semg0).wait()
    for g in range(_BPW // _LANES):
        sl = pl.ds(g * _LANES, _LANES)
        one = jnp.int32(1)
        zero = jnp.int32(0)
        p = jnp.zeros((_LANES,), jnp.int32)
        for l in range(_NDENSE):
            b = jnp.where(lam_all[l, sl] > 0.0, one, zero)
            p = p + (b << (_NDENSE - 1 - l))
        idxd = p + jnp.int32(_NDNODE - 1)
        for i in range(_NDEEP):
            if i > 0:
                b = jnp.where(lam_all[_NDENSE + i - 1, sl] > 0.0, one, zero)
                idxd = idxd * 2 + 1 + b
            dp_all[pl.ds(g * _LANES * _NDEEP + i * _LANES, _LANES)] = idxd

    def par_slice(c):
        return par_hbm.at[pl.ds(base + c * _CHUNK, _CHUNK)]

    def out_slice(c):
        return out_hbm.at[pl.ds(base + c * _CHUNK, _CHUNK)]

    _issue_gather(0, ytab_hbm, dp_all, rows_v0, semg0)
    pltpu.async_copy(par_slice(0), par_v0, semp0)

    def loop(c2, carry):
        c = c2 * 2
        _issue_gather(c + 1, ytab_hbm, dp_all, rows_v1, semg1)
        pltpu.async_copy(par_slice(c + 1), par_v1, semp1)
        _wait_gather(c, ytab_hbm, dp_all, rows_v0, semg0)
        pltpu.make_async_copy(par_slice(c), par_v0, semp0).wait()

        @pl.when(c2 > 0)
        def _():
            pltpu.make_async_copy(out_v0, out_slice(c - 2), semo0).wait()

        _compute_chunk(c, lam_all, rows_v0, par_v0, out_v0)
        pltpu.async_copy(out_v0, out_slice(c), semo0)

        @pl.when(c2 < _NCHUNKS // 2 - 1)
        def _():
            _issue_gather(c + 2, ytab_hbm, dp_all, rows_v0, semg0)
            pltpu.async_copy(par_slice(c + 2), par_v0, semp0)

        _wait_gather(c + 1, ytab_hbm, dp_all, rows_v1, semg1)
        pltpu.make_async_copy(par_slice(c + 1), par_v1, semp1).wait()

        @pl.when(c2 > 0)
        def _():
            pltpu.make_async_copy(out_v1, out_slice(c - 1), semo1).wait()

        _compute_chunk(c + 1, lam_all, rows_v1, par_v1, out_v1)
        pltpu.async_copy(out_v1, out_slice(c + 1), semo1)
        return carry

    lax.fori_loop(0, _NCHUNKS // 2, loop, 0)
    pltpu.make_async_copy(out_v0, out_slice(_NCHUNKS - 2), semo0).wait()
    pltpu.make_async_copy(out_v1, out_slice(_NCHUNKS - 1), semo1).wait()


def _combine(lamt, ytab_packed, partial):
    mesh = plsc.VectorSubcoreMesh(core_axis_name="c", subcore_axis_name="s",
                                  num_cores=_NC, num_subcores=_NS)
    f = pl.kernel(
        _combine_body,
        out_type=jax.ShapeDtypeStruct((_B, _NOUT), jnp.float32),
        mesh=mesh,
        scratch_types=[
            pltpu.VMEM((_DEPTH, _BPW), jnp.float32),
            pltpu.VMEM((_BPW * _NDEEP,), jnp.int32),
            pltpu.VMEM((_CGD, _HALF), jnp.int32),
            pltpu.VMEM((_CGD, _HALF), jnp.int32),
            pltpu.VMEM((_CHUNK, _HALF), jnp.int32),
            pltpu.VMEM((_CHUNK, _HALF), jnp.int32),
            pltpu.VMEM((_CHUNK, _NOUT), jnp.float32),
            pltpu.VMEM((_CHUNK, _NOUT), jnp.float32),
            pltpu.SemaphoreType.DMA,
            pltpu.SemaphoreType.DMA,
            pltpu.SemaphoreType.DMA,
            pltpu.SemaphoreType.DMA,
            pltpu.SemaphoreType.DMA,
            pltpu.SemaphoreType.DMA,
        ],
    )
    return f(lamt, ytab_packed, partial)


def kernel(x, W_sel, Y):
    orig_shape = x.shape
    x2 = x.reshape(-1, _NIN) if x.ndim == 3 else x
    partial, lamt, ytab = _router(x2, W_sel, Y)
    y = _combine(lamt, ytab, partial)
    if orig_shape[1] != _NIN:
        y = y.reshape(orig_shape[0], orig_shape[1], _NOUT)
    return y
